# baseline (speedup 1.0000x reference)
import functools

import jax
import jax.numpy as jnp
from jax import lax
from jax.experimental import pallas as pl
from jax.experimental.pallas import tpu as pltpu
from jax.experimental.pallas import tpu_sc as plsc

_NC, _NS, _L = 2, 16, 16
_NW = _NC * _NS
_D = 128
_DG = _D // _L


def _sc_mesh():
    return plsc.VectorSubcoreMesh(core_axis_name="c", subcore_axis_name="s")


def _rsqrt16(x):
    xi = plsc.bitcast(x, jnp.int32)
    yi = 0x5F3759DF - lax.shift_right_arithmetic(xi, 1)
    y = plsc.bitcast(yi, jnp.float32)
    for _ in range(3):
        y = y * (1.5 - 0.5 * x * y * y)
    return y


def _deg_partials(src, dst, w, n_pad):
    E = src.shape[0]
    e_per = E // _NW

    @functools.partial(
        pl.kernel,
        compiler_params=pltpu.CompilerParams(needs_layout_passes=False),
        out_type=jax.ShapeDtypeStruct((_NW, 2, n_pad), jnp.float32),
        mesh=_sc_mesh(),
        scratch_types=[
            pltpu.VMEM((e_per,), jnp.int32),
            pltpu.VMEM((e_per,), jnp.int32),
            pltpu.VMEM((e_per,), jnp.float32),
            pltpu.VMEM((n_pad,), jnp.float32),
            pltpu.VMEM((n_pad,), jnp.float32),
        ],
    )
    def k(src_h, dst_h, w_h, out_h, src_v, dst_v, w_v, ds_v, dd_v):
        c = lax.axis_index("c")
        s = lax.axis_index("s")
        wid = c * _NS + s
        base = wid * e_per
        z = jnp.zeros((_L,), jnp.float32)

        def zero(i, carry):
            ds_v[pl.ds(i * _L, _L)] = z
            dd_v[pl.ds(i * _L, _L)] = z
            return carry

        lax.fori_loop(0, n_pad // _L, zero, 0)
        pltpu.sync_copy(src_h.at[pl.ds(base, e_per)], src_v)
        pltpu.sync_copy(dst_h.at[pl.ds(base, e_per)], dst_v)
        pltpu.sync_copy(w_h.at[pl.ds(base, e_per)], w_v)

        def body(i, carry):
            sl = pl.ds(i * _L, _L)
            wv = w_v[sl]
            plsc.addupdate_scatter(ds_v, [src_v[sl]], wv)
            plsc.addupdate_scatter(dd_v, [dst_v[sl]], wv)
            return carry

        lax.fori_loop(0, e_per // _L, body, 0)
        pltpu.sync_copy(ds_v, out_h.at[wid, 0])
        pltpu.sync_copy(dd_v, out_h.at[wid, 1])

    return k(src, dst, w)


def _edge_norm(parts, src, dst, w, n_pad):
    E = src.shape[0]
    e_per = E // _NW
    npc = n_pad // _NS

    @functools.partial(
        pl.kernel,
        compiler_params=pltpu.CompilerParams(needs_layout_passes=False),
        out_type=jax.ShapeDtypeStruct((E,), jnp.float32),
        mesh=_sc_mesh(),
        scratch_types=[
            pltpu.VMEM((_NW, 2, npc), jnp.float32),
            pltpu.VMEM((2, npc), jnp.float32),
            pltpu.VMEM((n_pad,), jnp.float32),
            pltpu.VMEM((n_pad,), jnp.float32),
            pltpu.VMEM_SHARED((2, n_pad), jnp.float32),
            pltpu.VMEM((e_per,), jnp.int32),
            pltpu.VMEM((e_per,), jnp.int32),
            pltpu.VMEM((e_per,), jnp.float32),
            pltpu.VMEM((e_per,), jnp.float32),
            pltpu.SemaphoreType.DMA,
            pltpu.SemaphoreType.DMA,
        ],
    )
    def k(parts_h, src_h, dst_h, w_h, norm_h, stage_v, rsl_v, rss_v, rsd_v,
          rs_sh, src_v, dst_v, w_v, nrm_v, sem_p, sem_st):
        c = lax.axis_index("c")
        s = lax.axis_index("s")
        wid = c * _NS + s
        nbase = s * npc
        ebase = wid * e_per

        pltpu.async_copy(src_h.at[pl.ds(ebase, e_per)], src_v, sem_st)
        pltpu.async_copy(dst_h.at[pl.ds(ebase, e_per)], dst_v, sem_st)
        pltpu.async_copy(w_h.at[pl.ds(ebase, e_per)], w_v, sem_st)
        for p in range(_NW):
            pltpu.async_copy(parts_h.at[p, 0, pl.ds(nbase, npc)],
                             stage_v.at[p, 0], sem_p)
            pltpu.async_copy(parts_h.at[p, 1, pl.ds(nbase, npc)],
                             stage_v.at[p, 1], sem_p)
        for p in range(_NW):
            pltpu.make_async_copy(parts_h.at[p, 0, pl.ds(nbase, npc)],
                                  stage_v.at[p, 0], sem_p).wait()
            pltpu.make_async_copy(parts_h.at[p, 1, pl.ds(nbase, npc)],
                                  stage_v.at[p, 1], sem_p).wait()

        def red(i, carry):
            sl = pl.ds(i * _L, _L)

            def acc(p, ab):
                return (ab[0] + stage_v[p, 0, sl], ab[1] + stage_v[p, 1, sl])

            zz = jnp.zeros((_L,), jnp.float32)
            a, b = lax.fori_loop(0, _NW, acc, (zz, zz))
            rsl_v[0, sl] = _rsqrt16(jnp.maximum(a, 1e-6))
            rsl_v[1, sl] = _rsqrt16(jnp.maximum(b, 1e-6))
            return carry

        lax.fori_loop(0, npc // _L, red, 0)
        pltpu.sync_copy(rsl_v.at[0], rs_sh.at[0, pl.ds(nbase, npc)])
        pltpu.sync_copy(rsl_v.at[1], rs_sh.at[1, pl.ds(nbase, npc)])
        plsc.subcore_barrier()
        pltpu.sync_copy(rs_sh.at[0], rss_v)
        pltpu.sync_copy(rs_sh.at[1], rsd_v)
        pltpu.make_async_copy(src_h.at[pl.ds(ebase, e_per)], src_v,
                              sem_st).wait()
        pltpu.make_async_copy(dst_h.at[pl.ds(ebase, e_per)], dst_v,
                              sem_st).wait()
        pltpu.make_async_copy(w_h.at[pl.ds(ebase, e_per)], w_v,
                              sem_st).wait()

        def inner(i, carry):
            sl = pl.ds(i * _L, _L)
            a = plsc.load_gather(rss_v, [src_v[sl]])
            b = plsc.load_gather(rsd_v, [dst_v[sl]])
            nrm_v[sl] = w_v[sl] * a * b
            return carry

        lax.fori_loop(0, e_per // _L, inner, 0)
        pltpu.sync_copy(nrm_v, norm_h.at[pl.ds(ebase, e_per)])

    return k(parts, src, dst, w)


def _prop(h, src_flat, dst_flat, nrm_flat):
    N = h.shape[0]
    E = src_flat.shape[0]
    e_per = E // _NW
    C = 80
    nch = e_per // C
    rpt = N // _NS

    @functools.partial(
        pl.kernel,
        compiler_params=pltpu.CompilerParams(needs_layout_passes=False),
        out_type=jax.ShapeDtypeStruct((_NC, N, _D), jnp.float32),
        mesh=_sc_mesh(),
        scratch_types=[
            pltpu.VMEM_SHARED((N, _D), jnp.float32),
            pltpu.VMEM((3, C), jnp.int32),
            pltpu.VMEM((3, C), jnp.int32),
            pltpu.VMEM((3, C), jnp.float32),
            pltpu.VMEM((C, _D), jnp.float32),
            pltpu.VMEM((C, _D), jnp.float32),
            pltpu.VMEM((C, _D), jnp.float32),
            [pltpu.SemaphoreType.DMA] * 12,
        ],
    )
    def k(h_h, src_h, dst_h, nrm_h, out_h, acc_sh, sr_v, dr_v, nr_v,
          r0_v, r1_v, r2_v, sems):
        (sem_g0, sem_g1, sem_g2, sem_s0, sem_s1, sem_s2,
         sem_p0, sem_p1, sem_p2, sem_d0, sem_d1, sem_d2) = sems
        c = lax.axis_index("c")
        s = lax.axis_index("s")
        wid = c * _NS + s
        z = jnp.zeros((_L,), jnp.float32)
        ebase = wid * e_per

        def zb(r, carry):
            for j in range(_DG):
                r0_v[r, pl.ds(j * _L, _L)] = z
            return carry

        lax.fori_loop(0, C, zb, 0)
        rbase = s * rpt

        def zc(i, carry):
            pltpu.sync_copy(r0_v, acc_sh.at[pl.ds(rbase + i * C, C)])
            return carry

        lax.fori_loop(0, rpt // C, zc, 0)
        plsc.subcore_barrier()

        bufs = (r0_v, r1_v, r2_v)
        gsems = (sem_g0, sem_g1, sem_g2)
        ssems = (sem_s0, sem_s1, sem_s2)
        psems = (sem_p0, sem_p1, sem_p2)
        dsems = (sem_d0, sem_d1, sem_d2)

        def esl(j):
            return pl.ds(ebase + j * C, C)

        def spref(j, m):
            pltpu.async_copy(src_h.at[esl(j)], sr_v.at[m], psems[m])
            pltpu.async_copy(nrm_h.at[esl(j)], nr_v.at[m], psems[m])

        def wait_spref(j, m):
            pltpu.make_async_copy(src_h.at[esl(j)], sr_v.at[m],
                                  psems[m]).wait()
            pltpu.make_async_copy(nrm_h.at[esl(j)], nr_v.at[m],
                                  psems[m]).wait()

        def dpref(j, m):
            pltpu.async_copy(dst_h.at[esl(j)], dr_v.at[m], dsems[m])

        def wait_dpref(j, m):
            pltpu.make_async_copy(dst_h.at[esl(j)], dr_v.at[m],
                                  dsems[m]).wait()

        def gather(m):
            pltpu.async_copy(h_h.at[sr_v.at[m]], bufs[m], gsems[m])

        def wait_gather(m):
            pltpu.make_async_copy(h_h.at[sr_v.at[m]], bufs[m],
                                  gsems[m]).wait()

        def scatter(m):
            pltpu.async_copy(bufs[m], acc_sh.at[dr_v.at[m]], ssems[m],
                             add=True)

        def wait_scatter(m):
            pltpu.make_async_copy(bufs[m], acc_sh.at[dr_v.at[m]],
                                  ssems[m]).wait()

        def scale(m):
            rows = bufs[m]

            def grp(g, carry):
                nv = nr_v[m, pl.ds(g * _L, _L)]
                for e in range(_L):
                    nb = jnp.full((_L,), nv[e], jnp.float32)
                    r = g * _L + e
                    for q in range(_DG):
                        sl = pl.ds(q * _L, _L)
                        rows[r, sl] = rows[r, sl] * nb
                return carry

            lax.fori_loop(0, C // _L, grp, 0)

        def body(j, b):
            if j + 1 < nch:
                wait_spref(j + 1, (b + 1) % 3)
                gather((b + 1) % 3)
            wait_gather(b)
            scale(b)
            if j >= 1:
                wait_scatter((b + 2) % 3)
            wait_dpref(j, b)
            scatter(b)
            if j + 3 < nch:
                spref(j + 3, b)
            if j + 2 < nch:
                dpref(j + 2, (b + 2) % 3)

        spref(0, 0)
        spref(1, 1)
        spref(2, 2)
        dpref(0, 0)
        dpref(1, 1)
        wait_spref(0, 0)
        gather(0)

        def triple(t, carry):
            j0 = 3 * t
            for b in range(3):
                j = j0 + b

                def wrapped():
                    if b == 0:
                        @pl.when(j >= 1)
                        def _():
                            wait_scatter(2)

                        wait_spref(j + 1, 1)
                        gather(1)
                        wait_gather(0)
                        scale(0)
                        wait_dpref(j, 0)
                        scatter(0)

                        @pl.when(j + 3 < nch)
                        def _():
                            spref(j + 3, 0)

                        dpref(j + 2, 2)
                    else:
                        wait_spref(j + 1, (b + 1) % 3)
                        gather((b + 1) % 3)
                        wait_gather(b)
                        scale(b)
                        wait_scatter((b + 2) % 3)
                        wait_dpref(j, b)
                        scatter(b)

                        @pl.when(j + 3 < nch)
                        def _():
                            spref(j + 3, b)

                        @pl.when(j + 2 < nch)
                        def _():
                            dpref(j + 2, (b + 2) % 3)

                wrapped()
            return carry

        nfull = (nch - 2) // 3
        lax.fori_loop(0, nfull, triple, 0)
        for j in range(3 * nfull, nch):
            body(j, j % 3)
        wait_scatter((nch - 1) % 3)
        plsc.subcore_barrier()
        nwb = rpt // C

        def wb_slice(i):
            return pl.ds(rbase + i * C, C)

        pltpu.async_copy(acc_sh.at[wb_slice(0)], bufs[0], gsems[0])
        for i in range(nwb):
            b = i % 2
            pltpu.make_async_copy(acc_sh.at[wb_slice(i)], bufs[b],
                                  gsems[b]).wait()
            if i + 1 < nwb:
                if i >= 1:
                    pltpu.make_async_copy(bufs[1 - b],
                                          out_h.at[c, wb_slice(i - 1)],
                                          ssems[1 - b]).wait()
                pltpu.async_copy(acc_sh.at[wb_slice(i + 1)], bufs[1 - b],
                                 gsems[1 - b])
            pltpu.async_copy(bufs[b], out_h.at[c, wb_slice(i)], ssems[b])
        for i in (nwb - 2, nwb - 1):
            pltpu.make_async_copy(bufs[i % 2], out_h.at[c, wb_slice(i)],
                                  ssems[i % 2]).wait()

    return k(h, src_flat, dst_flat, nrm_flat)


def _pair_products(h, ps, pd, ns, nd):
    P = ps.shape[0]
    C = 80
    iters = P // (C * _NW)

    @functools.partial(
        pl.kernel,
        compiler_params=pltpu.CompilerParams(needs_layout_passes=False),
        out_type=jax.ShapeDtypeStruct((2 * P, _D), jnp.float32),
        mesh=_sc_mesh(),
        scratch_types=[
            pltpu.VMEM((2, C), jnp.int32),
            pltpu.VMEM((2, C), jnp.int32),
            pltpu.VMEM((2, C, _D), jnp.float32),
            pltpu.VMEM((2, C, _D), jnp.float32),
            [pltpu.SemaphoreType.DMA] * 6,
        ],
    )
    def k(h_h, ps_h, pd_h, ns_h, nd_h, z_h, a_v, b_v, ra_v, rb_v, sems):
        sem_i0, sem_i1, sem_a0, sem_a1, sem_o0, sem_o1 = sems
        c = lax.axis_index("c")
        s = lax.axis_index("s")
        wid = c * _NS + s
        isems = (sem_i0, sem_i1)
        gsems = (sem_a0, sem_a1)
        osems = (sem_o0, sem_o1)

        def do(pa_h, pb_h, obase):
            def cslice(t):
                return pl.ds((wid + t * _NW) * C, C)

            def pref(t, p):
                pltpu.async_copy(pa_h.at[cslice(t)], a_v.at[p], isems[p])
                pltpu.async_copy(pb_h.at[cslice(t)], b_v.at[p], isems[p])

            def wait_pref(t, p):
                pltpu.make_async_copy(pa_h.at[cslice(t)], a_v.at[p],
                                      isems[p]).wait()
                pltpu.make_async_copy(pb_h.at[cslice(t)], b_v.at[p],
                                      isems[p]).wait()

            def gath(p):
                pltpu.async_copy(h_h.at[a_v.at[p]], ra_v.at[p], gsems[p])
                pltpu.async_copy(h_h.at[b_v.at[p]], rb_v.at[p], gsems[p])

            def wait_gath(p):
                pltpu.make_async_copy(h_h.at[a_v.at[p]], ra_v.at[p],
                                      gsems[p]).wait()
                pltpu.make_async_copy(h_h.at[b_v.at[p]], rb_v.at[p],
                                      gsems[p]).wait()

            def owrite(t, p):
                pltpu.async_copy(ra_v.at[p],
                                 z_h.at[pl.ds(obase + (wid + t * _NW) * C, C)],
                                 osems[p])

            def wait_owrite(t, p):
                pltpu.make_async_copy(ra_v.at[p],
                                      z_h.at[pl.ds(obase + (wid + t * _NW) * C, C)],
                                      osems[p]).wait()

            pref(0, 0)
            pref(1, 1)
            wait_pref(0, 0)
            gath(0)
            for t in range(iters):
                p = t % 2
                if t + 1 < iters:
                    wait_pref(t + 1, 1 - p)
                    if t >= 1:
                        wait_owrite(t - 1, 1 - p)
                    gath(1 - p)
                wait_gath(p)
                if t + 2 < iters:
                    pref(t + 2, p)

                def mul(e, carry2):
                    for j in range(_DG):
                        sl = pl.ds(j * _L, _L)
                        ra_v[p, e, sl] = ra_v[p, e, sl] * rb_v[p, e, sl]
                    return carry2

                lax.fori_loop(0, C, mul, 0)
                owrite(t, p)
            wait_owrite(iters - 2, (iters - 2) % 2)
            wait_owrite(iters - 1, (iters - 1) % 2)

        do(ps_h, pd_h, 0)
        do(ns_h, nd_h, P)

    return k(h, ps, pd, ns, nd)


def _add_body(a_ref, b_ref, o_ref):
    o_ref[...] = a_ref[0] + b_ref[0]


def _combine(p):
    N = p.shape[1]
    blk = 2048
    return pl.pallas_call(
        _add_body,
        grid=(N // blk,),
        in_specs=[pl.BlockSpec((1, blk, _D), lambda i: (0, i, 0)),
                  pl.BlockSpec((1, blk, _D), lambda i: (1, i, 0))],
        out_specs=pl.BlockSpec((blk, _D), lambda i: (i, 0)),
        out_shape=jax.ShapeDtypeStruct((N, _D), jnp.float32),
    )(p, p)


def _tag_linear_body(h0, h1, h2, p3a, p3b, w_ref, b_ref, o_ref, *, relu):
    w = w_ref[...]
    acc = (h0[...] @ w[0:128]
           + h1[...] @ w[128:256]
           + h2[...] @ w[256:384]
           + (p3a[0] + p3b[0]) @ w[384:512]
           + b_ref[...])
    o_ref[...] = jnp.maximum(acc, 0.0) if relu else acc


def _tag_linear(h0, h1, h2, p3, W, b, relu):
    N = h0.shape[0]
    blk = 2048
    return pl.pallas_call(
        functools.partial(_tag_linear_body, relu=relu),
        grid=(N // blk,),
        in_specs=[pl.BlockSpec((blk, _D), lambda i: (i, 0))] * 3
        + [pl.BlockSpec((1, blk, _D), lambda i: (0, i, 0)),
           pl.BlockSpec((1, blk, _D), lambda i: (1, i, 0)),
           pl.BlockSpec((4 * _D, _D), lambda i: (0, 0)),
           pl.BlockSpec((_D,), lambda i: (0,))],
        out_specs=pl.BlockSpec((blk, _D), lambda i: (i, 0)),
        out_shape=jax.ShapeDtypeStruct((N, _D), jnp.float32),
    )(h0, h1, h2, p3, p3, W, b)


def _pred_body(z_ref, p1_ref, pb1_ref, p2_ref, pb2_ref, p3_ref, pb3_ref, o_ref):
    t = z_ref[...] @ p1_ref[...] + pb1_ref[...]
    t = jnp.where(t > 0, t, 0.2 * t)
    t = t @ p2_ref[...] + pb2_ref[...]
    t = jnp.where(t > 0, t, 0.2 * t)
    o_ref[...] = t @ p3_ref[...] + pb3_ref[...]


def _predictor(z, P1, pb1, P2, pb2, P3, pb3):
    B = z.shape[0]
    blk = 2048
    return pl.pallas_call(
        _pred_body,
        grid=(B // blk,),
        in_specs=[
            pl.BlockSpec((blk, _D), lambda i: (i, 0)),
            pl.BlockSpec((_D, 64), lambda i: (0, 0)),
            pl.BlockSpec((64,), lambda i: (0,)),
            pl.BlockSpec((64, 32), lambda i: (0, 0)),
            pl.BlockSpec((32,), lambda i: (0,)),
            pl.BlockSpec((32, 1), lambda i: (0, 0)),
            pl.BlockSpec((1,), lambda i: (0,)),
        ],
        out_specs=pl.BlockSpec((blk, 1), lambda i: (i, 0)),
        out_shape=jax.ShapeDtypeStruct((B, 1), jnp.float32),
    )(z, P1, pb1, P2, pb2, P3, pb3)


def kernel(x, edge_index, edge_weight, pos_edges, neg_edges,
           W1, b1, W2, b2, W3, b3, P1, pb1, P2, pb2, P3, pb3):
    N = x.shape[0]
    n_pad = ((N + _NW * _L - 1) // (_NW * _L)) * (_NW * _L)
    src = edge_index[0].astype(jnp.int32)
    dst = edge_index[1].astype(jnp.int32)
    w = edge_weight.astype(jnp.float32)

    parts = _deg_partials(src, dst, w, n_pad)
    norm = _edge_norm(parts, src, dst, w, n_pad)


    h = jnp.pad(x, ((0, n_pad - N), (0, 0)))
    for W, b, act in ((W1, b1, True), (W2, b2, True), (W3, b3, False)):
        f0 = h
        p1 = _prop(f0, src, dst, norm)
        h1 = _combine(p1)
        p2 = _prop(h1, src, dst, norm)
        h2 = _combine(p2)
        p3 = _prop(h2, src, dst, norm)
        h = _tag_linear(f0, h1, h2, p3, W, b, act)

    pe = pos_edges.astype(jnp.int32)
    ne = neg_edges.astype(jnp.int32)
    P = pe.shape[1]
    p_pad = ((P + 80 * _NW - 1) // (80 * _NW)) * (80 * _NW)
    pad = ((0, p_pad - P),)
    z = _pair_products(h, jnp.pad(pe[0], pad), jnp.pad(pe[1], pad),
                       jnp.pad(ne[0], pad), jnp.pad(ne[1], pad))
    hz = _predictor(z, P1, pb1, P2, pb2, P3, pb3)
    return (hz[:P], hz[p_pad:p_pad + P], h[:N])


# speedup vs baseline: 1.0029x; 1.0029x over previous
import functools

import jax
import jax.numpy as jnp
from jax import lax
from jax.experimental import pallas as pl
from jax.experimental.pallas import tpu as pltpu
from jax.experimental.pallas import tpu_sc as plsc

_NC, _NS, _L = 2, 16, 16
_NW = _NC * _NS
_D = 128
_DG = _D // _L


def _sc_mesh():
    return plsc.VectorSubcoreMesh(core_axis_name="c", subcore_axis_name="s")


def _rsqrt16(x):
    xi = plsc.bitcast(x, jnp.int32)
    yi = 0x5F3759DF - lax.shift_right_arithmetic(xi, 1)
    y = plsc.bitcast(yi, jnp.float32)
    for _ in range(3):
        y = y * (1.5 - 0.5 * x * y * y)
    return y


def _deg_partials(src, dst, w, n_pad):
    E = src.shape[0]
    e_per = E // _NW

    @functools.partial(
        pl.kernel,
        compiler_params=pltpu.CompilerParams(needs_layout_passes=False),
        out_type=jax.ShapeDtypeStruct((_NW, 2, n_pad), jnp.float32),
        mesh=_sc_mesh(),
        scratch_types=[
            pltpu.VMEM((e_per,), jnp.int32),
            pltpu.VMEM((e_per,), jnp.int32),
            pltpu.VMEM((e_per,), jnp.float32),
            pltpu.VMEM((n_pad,), jnp.float32),
            pltpu.VMEM((n_pad,), jnp.float32),
        ],
    )
    def k(src_h, dst_h, w_h, out_h, src_v, dst_v, w_v, ds_v, dd_v):
        c = lax.axis_index("c")
        s = lax.axis_index("s")
        wid = c * _NS + s
        base = wid * e_per
        z = jnp.zeros((_L,), jnp.float32)

        def zero(i, carry):
            ds_v[pl.ds(i * _L, _L)] = z
            dd_v[pl.ds(i * _L, _L)] = z
            return carry

        lax.fori_loop(0, n_pad // _L, zero, 0)
        pltpu.sync_copy(src_h.at[pl.ds(base, e_per)], src_v)
        pltpu.sync_copy(dst_h.at[pl.ds(base, e_per)], dst_v)
        pltpu.sync_copy(w_h.at[pl.ds(base, e_per)], w_v)

        def body(i, carry):
            sl = pl.ds(i * _L, _L)
            wv = w_v[sl]
            plsc.addupdate_scatter(ds_v, [src_v[sl]], wv)
            plsc.addupdate_scatter(dd_v, [dst_v[sl]], wv)
            return carry

        lax.fori_loop(0, e_per // _L, body, 0)
        pltpu.sync_copy(ds_v, out_h.at[wid, 0])
        pltpu.sync_copy(dd_v, out_h.at[wid, 1])

    return k(src, dst, w)


def _edge_norm(parts, src, dst, w, n_pad):
    E = src.shape[0]
    e_per = E // _NW
    npc = n_pad // _NS

    @functools.partial(
        pl.kernel,
        compiler_params=pltpu.CompilerParams(needs_layout_passes=False),
        out_type=jax.ShapeDtypeStruct((E,), jnp.float32),
        mesh=_sc_mesh(),
        scratch_types=[
            pltpu.VMEM((_NW, 2, npc), jnp.float32),
            pltpu.VMEM((2, npc), jnp.float32),
            pltpu.VMEM((n_pad,), jnp.float32),
            pltpu.VMEM((n_pad,), jnp.float32),
            pltpu.VMEM_SHARED((2, n_pad), jnp.float32),
            pltpu.VMEM((e_per,), jnp.int32),
            pltpu.VMEM((e_per,), jnp.int32),
            pltpu.VMEM((e_per,), jnp.float32),
            pltpu.VMEM((e_per,), jnp.float32),
            pltpu.SemaphoreType.DMA,
            pltpu.SemaphoreType.DMA,
        ],
    )
    def k(parts_h, src_h, dst_h, w_h, norm_h, stage_v, rsl_v, rss_v, rsd_v,
          rs_sh, src_v, dst_v, w_v, nrm_v, sem_p, sem_st):
        c = lax.axis_index("c")
        s = lax.axis_index("s")
        wid = c * _NS + s
        nbase = s * npc
        ebase = wid * e_per

        pltpu.async_copy(src_h.at[pl.ds(ebase, e_per)], src_v, sem_st)
        pltpu.async_copy(dst_h.at[pl.ds(ebase, e_per)], dst_v, sem_st)
        pltpu.async_copy(w_h.at[pl.ds(ebase, e_per)], w_v, sem_st)
        for p in range(_NW):
            pltpu.async_copy(parts_h.at[p, 0, pl.ds(nbase, npc)],
                             stage_v.at[p, 0], sem_p)
            pltpu.async_copy(parts_h.at[p, 1, pl.ds(nbase, npc)],
                             stage_v.at[p, 1], sem_p)
        for p in range(_NW):
            pltpu.make_async_copy(parts_h.at[p, 0, pl.ds(nbase, npc)],
                                  stage_v.at[p, 0], sem_p).wait()
            pltpu.make_async_copy(parts_h.at[p, 1, pl.ds(nbase, npc)],
                                  stage_v.at[p, 1], sem_p).wait()

        def red(i, carry):
            sl = pl.ds(i * _L, _L)

            def acc(p, ab):
                return (ab[0] + stage_v[p, 0, sl], ab[1] + stage_v[p, 1, sl])

            zz = jnp.zeros((_L,), jnp.float32)
            a, b = lax.fori_loop(0, _NW, acc, (zz, zz))
            rsl_v[0, sl] = _rsqrt16(jnp.maximum(a, 1e-6))
            rsl_v[1, sl] = _rsqrt16(jnp.maximum(b, 1e-6))
            return carry

        lax.fori_loop(0, npc // _L, red, 0)
        pltpu.sync_copy(rsl_v.at[0], rs_sh.at[0, pl.ds(nbase, npc)])
        pltpu.sync_copy(rsl_v.at[1], rs_sh.at[1, pl.ds(nbase, npc)])
        plsc.subcore_barrier()
        pltpu.sync_copy(rs_sh.at[0], rss_v)
        pltpu.sync_copy(rs_sh.at[1], rsd_v)
        pltpu.make_async_copy(src_h.at[pl.ds(ebase, e_per)], src_v,
                              sem_st).wait()
        pltpu.make_async_copy(dst_h.at[pl.ds(ebase, e_per)], dst_v,
                              sem_st).wait()
        pltpu.make_async_copy(w_h.at[pl.ds(ebase, e_per)], w_v,
                              sem_st).wait()

        def inner(i, carry):
            sl = pl.ds(i * _L, _L)
            a = plsc.load_gather(rss_v, [src_v[sl]])
            b = plsc.load_gather(rsd_v, [dst_v[sl]])
            nrm_v[sl] = w_v[sl] * a * b
            return carry

        lax.fori_loop(0, e_per // _L, inner, 0)
        pltpu.sync_copy(nrm_v, norm_h.at[pl.ds(ebase, e_per)])

    return k(parts, src, dst, w)


def _prop(h, src_flat, dst_flat, nrm_flat, n_acc):
    N = n_acc
    E = src_flat.shape[0]
    e_per = E // _NW
    C = 80
    nch = e_per // C
    rpt = N // _NS

    @functools.partial(
        pl.kernel,
        compiler_params=pltpu.CompilerParams(needs_layout_passes=False),
        out_type=jax.ShapeDtypeStruct((_NC, N, _D), jnp.float32),
        mesh=_sc_mesh(),
        scratch_types=[
            pltpu.VMEM_SHARED((N, _D), jnp.float32),
            pltpu.VMEM((3, C), jnp.int32),
            pltpu.VMEM((3, C), jnp.int32),
            pltpu.VMEM((3, C), jnp.float32),
            pltpu.VMEM((C, _D), jnp.float32),
            pltpu.VMEM((C, _D), jnp.float32),
            pltpu.VMEM((C, _D), jnp.float32),
            [pltpu.SemaphoreType.DMA] * 12,
        ],
    )
    def k(h_h, src_h, dst_h, nrm_h, out_h, acc_sh, sr_v, dr_v, nr_v,
          r0_v, r1_v, r2_v, sems):
        (sem_g0, sem_g1, sem_g2, sem_s0, sem_s1, sem_s2,
         sem_p0, sem_p1, sem_p2, sem_d0, sem_d1, sem_d2) = sems
        c = lax.axis_index("c")
        s = lax.axis_index("s")
        wid = c * _NS + s
        z = jnp.zeros((_L,), jnp.float32)
        ebase = wid * e_per

        def zb(r, carry):
            for j in range(_DG):
                r0_v[r, pl.ds(j * _L, _L)] = z
            return carry

        lax.fori_loop(0, C, zb, 0)
        rbase = s * rpt

        def zc(i, carry):
            pltpu.sync_copy(r0_v, acc_sh.at[pl.ds(rbase + i * C, C)])
            return carry

        lax.fori_loop(0, rpt // C, zc, 0)
        plsc.subcore_barrier()

        bufs = (r0_v, r1_v, r2_v)
        gsems = (sem_g0, sem_g1, sem_g2)
        ssems = (sem_s0, sem_s1, sem_s2)
        psems = (sem_p0, sem_p1, sem_p2)
        dsems = (sem_d0, sem_d1, sem_d2)

        def esl(j):
            return pl.ds(ebase + j * C, C)

        def spref(j, m):
            pltpu.async_copy(src_h.at[esl(j)], sr_v.at[m], psems[m])
            pltpu.async_copy(nrm_h.at[esl(j)], nr_v.at[m], psems[m])

        def wait_spref(j, m):
            pltpu.make_async_copy(src_h.at[esl(j)], sr_v.at[m],
                                  psems[m]).wait()
            pltpu.make_async_copy(nrm_h.at[esl(j)], nr_v.at[m],
                                  psems[m]).wait()

        def dpref(j, m):
            pltpu.async_copy(dst_h.at[esl(j)], dr_v.at[m], dsems[m])

        def wait_dpref(j, m):
            pltpu.make_async_copy(dst_h.at[esl(j)], dr_v.at[m],
                                  dsems[m]).wait()

        def gather(m):
            pltpu.async_copy(h_h.at[sr_v.at[m]], bufs[m], gsems[m])

        def wait_gather(m):
            pltpu.make_async_copy(h_h.at[sr_v.at[m]], bufs[m],
                                  gsems[m]).wait()

        def scatter(m):
            pltpu.async_copy(bufs[m], acc_sh.at[dr_v.at[m]], ssems[m],
                             add=True)

        def wait_scatter(m):
            pltpu.make_async_copy(bufs[m], acc_sh.at[dr_v.at[m]],
                                  ssems[m]).wait()

        def scale(m):
            rows = bufs[m]

            def grp(g, carry):
                nv = nr_v[m, pl.ds(g * _L, _L)]
                for e in range(_L):
                    nb = jnp.full((_L,), nv[e], jnp.float32)
                    r = g * _L + e
                    for q in range(_DG):
                        sl = pl.ds(q * _L, _L)
                        rows[r, sl] = rows[r, sl] * nb
                return carry

            lax.fori_loop(0, C // _L, grp, 0)

        def body(j, b):
            if j + 1 < nch:
                wait_spref(j + 1, (b + 1) % 3)
                gather((b + 1) % 3)
            wait_gather(b)
            scale(b)
            if j >= 1:
                wait_scatter((b + 2) % 3)
            wait_dpref(j, b)
            scatter(b)
            if j + 3 < nch:
                spref(j + 3, b)
            if j + 2 < nch:
                dpref(j + 2, (b + 2) % 3)

        spref(0, 0)
        spref(1, 1)
        spref(2, 2)
        dpref(0, 0)
        dpref(1, 1)
        wait_spref(0, 0)
        gather(0)

        def triple(t, carry):
            j0 = 3 * t
            for b in range(3):
                j = j0 + b

                def wrapped():
                    if b == 0:
                        @pl.when(j >= 1)
                        def _():
                            wait_scatter(2)

                        wait_spref(j + 1, 1)
                        gather(1)
                        wait_gather(0)
                        scale(0)
                        wait_dpref(j, 0)
                        scatter(0)

                        @pl.when(j + 3 < nch)
                        def _():
                            spref(j + 3, 0)

                        dpref(j + 2, 2)
                    else:
                        wait_spref(j + 1, (b + 1) % 3)
                        gather((b + 1) % 3)
                        wait_gather(b)
                        scale(b)
                        wait_scatter((b + 2) % 3)
                        wait_dpref(j, b)
                        scatter(b)

                        @pl.when(j + 3 < nch)
                        def _():
                            spref(j + 3, b)

                        @pl.when(j + 2 < nch)
                        def _():
                            dpref(j + 2, (b + 2) % 3)

                wrapped()
            return carry

        nfull = (nch - 2) // 3
        lax.fori_loop(0, nfull, triple, 0)
        for j in range(3 * nfull, nch):
            body(j, j % 3)
        wait_scatter((nch - 1) % 3)
        plsc.subcore_barrier()
        nwb = rpt // C

        def wb_slice(i):
            return pl.ds(rbase + i * C, C)

        pltpu.async_copy(acc_sh.at[wb_slice(0)], bufs[0], gsems[0])
        for i in range(nwb):
            b = i % 2
            pltpu.make_async_copy(acc_sh.at[wb_slice(i)], bufs[b],
                                  gsems[b]).wait()
            if i + 1 < nwb:
                if i >= 1:
                    pltpu.make_async_copy(bufs[1 - b],
                                          out_h.at[c, wb_slice(i - 1)],
                                          ssems[1 - b]).wait()
                pltpu.async_copy(acc_sh.at[wb_slice(i + 1)], bufs[1 - b],
                                 gsems[1 - b])
            pltpu.async_copy(bufs[b], out_h.at[c, wb_slice(i)], ssems[b])
        for i in (nwb - 2, nwb - 1):
            pltpu.make_async_copy(bufs[i % 2], out_h.at[c, wb_slice(i)],
                                  ssems[i % 2]).wait()

    return k(h, src_flat, dst_flat, nrm_flat)


def _pair_products(h, ps, pd, ns, nd):
    P = ps.shape[0]
    C = 80
    iters = P // (C * _NW)

    @functools.partial(
        pl.kernel,
        compiler_params=pltpu.CompilerParams(needs_layout_passes=False),
        out_type=jax.ShapeDtypeStruct((2 * P, _D), jnp.float32),
        mesh=_sc_mesh(),
        scratch_types=[
            pltpu.VMEM((2, C), jnp.int32),
            pltpu.VMEM((2, C), jnp.int32),
            pltpu.VMEM((C, _D), jnp.float32),
            pltpu.VMEM((C, _D), jnp.float32),
            pltpu.VMEM((C, _D), jnp.float32),
            pltpu.VMEM((C, _D), jnp.float32),
            [pltpu.SemaphoreType.DMA] * 6,
        ],
    )
    def k(h_h, ps_h, pd_h, ns_h, nd_h, z_h, a_v, b_v, ra0_v, ra1_v,
          rb0_v, rb1_v, sems):
        sem_i0, sem_i1, sem_a0, sem_a1, sem_o0, sem_o1 = sems
        ras = (ra0_v, ra1_v)
        rbs = (rb0_v, rb1_v)
        c = lax.axis_index("c")
        s = lax.axis_index("s")
        wid = c * _NS + s
        isems = (sem_i0, sem_i1)
        gsems = (sem_a0, sem_a1)
        osems = (sem_o0, sem_o1)

        def do(pa_h, pb_h, obase):
            def cslice(t):
                return pl.ds((wid + t * _NW) * C, C)

            def pref(t, p):
                pltpu.async_copy(pa_h.at[cslice(t)], a_v.at[p], isems[p])
                pltpu.async_copy(pb_h.at[cslice(t)], b_v.at[p], isems[p])

            def wait_pref(t, p):
                pltpu.make_async_copy(pa_h.at[cslice(t)], a_v.at[p],
                                      isems[p]).wait()
                pltpu.make_async_copy(pb_h.at[cslice(t)], b_v.at[p],
                                      isems[p]).wait()

            def gath(p):
                pltpu.async_copy(h_h.at[a_v.at[p]], ras[p], gsems[p])
                pltpu.async_copy(h_h.at[b_v.at[p]], rbs[p], gsems[p])

            def wait_gath(p):
                pltpu.make_async_copy(h_h.at[a_v.at[p]], ras[p],
                                      gsems[p]).wait()
                pltpu.make_async_copy(h_h.at[b_v.at[p]], rbs[p],
                                      gsems[p]).wait()

            def owrite(t, p):
                pltpu.async_copy(ras[p],
                                 z_h.at[pl.ds(obase + (wid + t * _NW) * C, C)],
                                 osems[p])

            def wait_owrite(t, p):
                pltpu.make_async_copy(ras[p],
                                      z_h.at[pl.ds(obase + (wid + t * _NW) * C, C)],
                                      osems[p]).wait()

            pref(0, 0)
            pref(1, 1)
            wait_pref(0, 0)
            gath(0)
            for t in range(iters):
                p = t % 2
                if t + 1 < iters:
                    wait_pref(t + 1, 1 - p)
                    if t >= 1:
                        wait_owrite(t - 1, 1 - p)
                    gath(1 - p)
                wait_gath(p)
                if t + 2 < iters:
                    pref(t + 2, p)

                ra, rb = ras[p], rbs[p]

                def mul(e, carry2):
                    for j in range(_DG):
                        sl = pl.ds(j * _L, _L)
                        ra[e, sl] = ra[e, sl] * rb[e, sl]
                    return carry2

                lax.fori_loop(0, C, mul, 0)
                owrite(t, p)
            wait_owrite(iters - 2, (iters - 2) % 2)
            wait_owrite(iters - 1, (iters - 1) % 2)

        do(ps_h, pd_h, 0)
        do(ns_h, nd_h, P)

    return k(h, ps, pd, ns, nd)


def _add_body(a_ref, b_ref, o_ref):
    o_ref[...] = a_ref[0] + b_ref[0]


def _combine(p, n_out):
    blk = 2000
    return pl.pallas_call(
        _add_body,
        grid=(n_out // blk,),
        in_specs=[pl.BlockSpec((1, blk, _D), lambda i: (0, i, 0)),
                  pl.BlockSpec((1, blk, _D), lambda i: (1, i, 0))],
        out_specs=pl.BlockSpec((blk, _D), lambda i: (i, 0)),
        out_shape=jax.ShapeDtypeStruct((n_out, _D), jnp.float32),
    )(p, p)


def _tag_linear_body(h0, h1, h2, p3a, p3b, w_ref, b_ref, o_ref, *, relu):
    w = w_ref[...]
    acc = (h0[...] @ w[0:128]
           + h1[...] @ w[128:256]
           + h2[...] @ w[256:384]
           + (p3a[0] + p3b[0]) @ w[384:512]
           + b_ref[...])
    o_ref[...] = jnp.maximum(acc, 0.0) if relu else acc


def _tag_linear(h0, h1, h2, p3, W, b, relu):
    N = h0.shape[0]
    blk = 2000
    return pl.pallas_call(
        functools.partial(_tag_linear_body, relu=relu),
        grid=(N // blk,),
        in_specs=[pl.BlockSpec((blk, _D), lambda i: (i, 0))] * 3
        + [pl.BlockSpec((1, blk, _D), lambda i: (0, i, 0)),
           pl.BlockSpec((1, blk, _D), lambda i: (1, i, 0)),
           pl.BlockSpec((4 * _D, _D), lambda i: (0, 0)),
           pl.BlockSpec((_D,), lambda i: (0,))],
        out_specs=pl.BlockSpec((blk, _D), lambda i: (i, 0)),
        out_shape=jax.ShapeDtypeStruct((N, _D), jnp.float32),
    )(h0, h1, h2, p3, p3, W, b)


def _pred_body(z_ref, p1_ref, pb1_ref, p2_ref, pb2_ref, p3_ref, pb3_ref, o_ref):
    t = z_ref[...] @ p1_ref[...] + pb1_ref[...]
    t = jnp.where(t > 0, t, 0.2 * t)
    t = t @ p2_ref[...] + pb2_ref[...]
    t = jnp.where(t > 0, t, 0.2 * t)
    o_ref[...] = t @ p3_ref[...] + pb3_ref[...]


def _predictor(z, P1, pb1, P2, pb2, P3, pb3):
    B = z.shape[0]
    blk = 2048
    return pl.pallas_call(
        _pred_body,
        grid=(B // blk,),
        in_specs=[
            pl.BlockSpec((blk, _D), lambda i: (i, 0)),
            pl.BlockSpec((_D, 64), lambda i: (0, 0)),
            pl.BlockSpec((64,), lambda i: (0,)),
            pl.BlockSpec((64, 32), lambda i: (0, 0)),
            pl.BlockSpec((32,), lambda i: (0,)),
            pl.BlockSpec((32, 1), lambda i: (0, 0)),
            pl.BlockSpec((1,), lambda i: (0,)),
        ],
        out_specs=pl.BlockSpec((blk, 1), lambda i: (i, 0)),
        out_shape=jax.ShapeDtypeStruct((B, 1), jnp.float32),
    )(z, P1, pb1, P2, pb2, P3, pb3)


def kernel(x, edge_index, edge_weight, pos_edges, neg_edges,
           W1, b1, W2, b2, W3, b3, P1, pb1, P2, pb2, P3, pb3):
    N = x.shape[0]
    n_pad = ((N + _NW * _L - 1) // (_NW * _L)) * (_NW * _L)
    src = edge_index[0].astype(jnp.int32)
    dst = edge_index[1].astype(jnp.int32)
    w = edge_weight.astype(jnp.float32)

    parts = _deg_partials(src, dst, w, n_pad)
    norm = _edge_norm(parts, src, dst, w, n_pad)


    h = x
    for W, b, act in ((W1, b1, True), (W2, b2, True), (W3, b3, False)):
        f0 = h
        p1 = _prop(f0, src, dst, norm, n_pad)
        h1 = _combine(p1, N)
        p2 = _prop(h1, src, dst, norm, n_pad)
        h2 = _combine(p2, N)
        p3 = _prop(h2, src, dst, norm, n_pad)
        h = _tag_linear(f0, h1, h2, p3, W, b, act)

    pe = pos_edges.astype(jnp.int32)
    ne = neg_edges.astype(jnp.int32)
    P = pe.shape[1]
    p_pad = ((P + 80 * _NW - 1) // (80 * _NW)) * (80 * _NW)
    pad = ((0, p_pad - P),)
    z = _pair_products(h, jnp.pad(pe[0], pad), jnp.pad(pe[1], pad),
                       jnp.pad(ne[0], pad), jnp.pad(ne[1], pad))
    hz = _predictor(z, P1, pb1, P2, pb2, P3, pb3)
    return (hz[:P], hz[p_pad:p_pad + P], h)
